# baseline (speedup 1.0000x reference)
import math

import jax
import jax.numpy as jnp
from jax.experimental import pallas as pl
from jax.experimental.pallas import tpu as pltpu

_INV_SQRT2 = 1.0 / math.sqrt(2.0)


def _gelu(x):
    return 0.5 * x * (1.0 + jax.lax.erf(x * _INV_SQRT2))


def _pack_w(weight):
    return jnp.concatenate(
        [weight[:, :, i, j] for i in range(2) for j in range(2)], axis=1
    )


def _up4_kernel(x_ref, w1_ref, b1_ref, w2_ref, b2_ref, o_ref):
    c_mid = w2_ref.shape[0]
    n2 = w2_ref.shape[1]
    x = x_ref[...].astype(jnp.bfloat16)
    y1 = jnp.dot(x, w1_ref[...], preferred_element_type=jnp.float32)
    y1 = _gelu(y1 + b1_ref[...])
    w2 = w2_ref[...]
    b2 = b2_ref[...]
    for t in range(4):
        y1t = y1[:, t * c_mid:(t + 1) * c_mid].astype(jnp.bfloat16)
        y2 = jnp.dot(y1t, w2, preferred_element_type=jnp.float32) + b2
        o_ref[:, t * n2:(t + 1) * n2] = y2


def _up4(x_nchw, w1, b1, w2, b2, bn_scale, bn_shift):
    N, C, H, W = x_nchw.shape
    C1 = w1.shape[1]
    Cout = w2.shape[1]
    M = N * H * W

    x_flat = jnp.transpose(x_nchw, (0, 2, 3, 1)).reshape(M, C)

    w1p = _pack_w(w1)
    s4 = jnp.tile(bn_scale, 4)[None, :]
    t4 = jnp.tile(bn_shift, 4)[None, :]
    w1p = (w1p * s4).astype(jnp.bfloat16)
    b1r = jnp.tile(b1, 4)[None, :] * s4 + t4
    w2p = _pack_w(w2).astype(jnp.bfloat16)
    b2r = jnp.tile(b2, 4)[None, :]

    n_out = 16 * Cout
    tm = 448
    grid = (pl.cdiv(M, tm),)

    y = pl.pallas_call(
        _up4_kernel,
        out_shape=jax.ShapeDtypeStruct((M, n_out), jnp.float32),
        grid=grid,
        in_specs=[
            pl.BlockSpec((tm, C), lambda i: (i, 0)),
            pl.BlockSpec((C, 4 * C1), lambda i: (0, 0)),
            pl.BlockSpec((1, 4 * C1), lambda i: (0, 0)),
            pl.BlockSpec((C1, 4 * Cout), lambda i: (0, 0)),
            pl.BlockSpec((1, 4 * Cout), lambda i: (0, 0)),
        ],
        out_specs=pl.BlockSpec((tm, n_out), lambda i: (i, 0)),
        compiler_params=pltpu.CompilerParams(
            dimension_semantics=("parallel",),
            vmem_limit_bytes=64 * 1024 * 1024,
        ),
    )(x_flat, w1p, b1r, w2p, b2r)

    y = y.reshape(N, H, W, 2, 2, 2, 2, Cout)
    y = jnp.transpose(y, (0, 7, 1, 3, 5, 2, 4, 6)).reshape(N, Cout, 4 * H, 4 * W)
    return y


def _convt_kernel(x_ref, w_ref, b_ref, o_ref):
    x = x_ref[...].astype(jnp.bfloat16)
    y = jnp.dot(x, w_ref[...], preferred_element_type=jnp.float32)
    o_ref[...] = y + b_ref[...]


def _up2(x_nchw, weight, bias):
    N, C, H, W = x_nchw.shape
    Cout = weight.shape[1]
    M = N * H * W

    x_flat = jnp.transpose(x_nchw, (0, 2, 3, 1)).reshape(M, C)
    w_packed = _pack_w(weight).astype(jnp.bfloat16)
    b_row = jnp.tile(bias, 4)[None, :]

    n_out = 4 * Cout
    tm = 448
    grid = (pl.cdiv(M, tm),)

    y = pl.pallas_call(
        _convt_kernel,
        out_shape=jax.ShapeDtypeStruct((M, n_out), jnp.float32),
        grid=grid,
        in_specs=[
            pl.BlockSpec((tm, C), lambda i: (i, 0)),
            pl.BlockSpec((C, n_out), lambda i: (0, 0)),
            pl.BlockSpec((1, n_out), lambda i: (0, 0)),
        ],
        out_specs=pl.BlockSpec((tm, n_out), lambda i: (i, 0)),
        compiler_params=pltpu.CompilerParams(
            dimension_semantics=("parallel",),
        ),
    )(x_flat, w_packed, b_row)

    y = y.reshape(N, H, W, 2, 2, Cout)
    y = jnp.transpose(y, (0, 5, 1, 3, 2, 4)).reshape(N, Cout, 2 * H, 2 * W)
    return y


def _max4_kernel(a_ref, b_ref, c_ref, d_ref, o_ref):
    o_ref[...] = jnp.maximum(
        jnp.maximum(a_ref[...], b_ref[...]), jnp.maximum(c_ref[...], d_ref[...])
    )


def _maxpool(x_nchw):
    N, C, H, W = x_nchw.shape
    H2, W2 = H // 2, W // 2

    a = x_nchw[:, :, 0::2, 0::2]
    b = x_nchw[:, :, 0::2, 1::2]
    c = x_nchw[:, :, 1::2, 0::2]
    d = x_nchw[:, :, 1::2, 1::2]

    numel = N * C * H2 * W2
    rows, cols = numel // 128, 128
    shape2d = (rows, cols)

    tr = 1024
    grid = (pl.cdiv(rows, tr),)
    spec = pl.BlockSpec((tr, cols), lambda i: (i, 0))

    out2d = pl.pallas_call(
        _max4_kernel,
        out_shape=jax.ShapeDtypeStruct(shape2d, x_nchw.dtype),
        grid=grid,
        in_specs=[spec, spec, spec, spec],
        out_specs=spec,
        compiler_params=pltpu.CompilerParams(
            dimension_semantics=("parallel",),
        ),
    )(a.reshape(shape2d), b.reshape(shape2d), c.reshape(shape2d), d.reshape(shape2d))
    return out2d.reshape(N, C, H2, W2)


def kernel(x0, x1, x2, x3, w4a, b4a, w4b, b4b, w2, b2, bn_scale, bn_shift):
    out0 = _up4(x0, w4a, b4a, w4b, b4b, bn_scale, bn_shift)
    out1 = _up2(x1, w2, b2)
    out3 = _maxpool(x3)
    return (out0, out1, x2, out3)
